# Initial kernel scaffold; baseline (speedup 1.0000x reference)
import jax
import jax.numpy as jnp
from jax.experimental import pallas as pl


def kernel(x, edge_index, regression_mask, W_self0, W_neigh0, b_conv0, bn_g0, bn_b0, W_self1, W_neigh1, b_conv1, bn_g1, bn_b1, W_lin0, b_lin0, ln_g0, ln_b0, W_lin1, b_lin1, ln_g1, ln_b1, W_head, b_head):
    raise NotImplementedError("write your pallas kernel here")


# speedup vs baseline: 6.2909x; 6.2909x over previous
import functools

import jax
import jax.numpy as jnp
from jax import lax
from jax.experimental import pallas as pl
from jax.experimental.pallas import tpu as pltpu
from jax.experimental.pallas import tpu_sc as plsc

N = 10000
D = 128
E = 320000
EPS = 1e-5

NC = 2
NS = 16
NW = NC * NS
EPW = E // NW
K = 80
CHUNKS = EPW // K
PAIRS = (CHUNKS - 1) // 2
N_PAD = 10240
ROWS_PT = N_PAD // NS

assert EPW % K == 0 and CHUNKS == 2 * PAIRS + 1
assert N_PAD % (8 * NS) == 0


def _sc_agg_body(with_deg, h_hbm, src_hbm, dst_hbm, z128_hbm, agg_out,
                 deg_out, agg_sh, sidx0, didx0, rows0, sidx1, didx1, rows1,
                 sem0, sem1):
    cid = lax.axis_index("c")
    sid = lax.axis_index("s")
    wid = sid * NC + cid
    r0 = sid * ROWS_PT
    ebase = wid * EPW

    pltpu.sync_copy(z128_hbm.at[pl.ds(r0, ROWS_PT)],
                    agg_sh.at[pl.ds(r0, ROWS_PT)])
    plsc.subcore_barrier()

    def load_and_fire(sidx, didx, rows, sem, off):
        pltpu.sync_copy(src_hbm.at[pl.ds(off, K)], sidx)
        pltpu.sync_copy(dst_hbm.at[pl.ds(off, K)], didx)
        return pltpu.async_copy(h_hbm.at[sidx], rows, sem)

    def drain_and_scatter(desc, didx, rows):
        desc.wait()
        pltpu.sync_copy(rows, agg_sh.at[didx], add=True)

    d0 = load_and_fire(sidx0, didx0, rows0, sem0, ebase)

    def pair(jj, carry):
        d1 = load_and_fire(sidx1, didx1, rows1, sem1,
                           ebase + (2 * jj + 1) * K)
        drain_and_scatter(d0, didx0, rows0)
        load_and_fire(sidx0, didx0, rows0, sem0, ebase + (2 * jj + 2) * K)
        drain_and_scatter(d1, didx1, rows1)
        return carry

    lax.fori_loop(0, PAIRS, pair, 0)
    drain_and_scatter(d0, didx0, rows0)

    plsc.subcore_barrier()
    pltpu.sync_copy(agg_sh.at[pl.ds(r0, ROWS_PT)],
                    agg_out.at[cid, pl.ds(r0, ROWS_PT)])

    if with_deg:
        plsc.subcore_barrier()
        pltpu.sync_copy(z128_hbm.at[pl.ds(r0, ROWS_PT)],
                        agg_sh.at[pl.ds(r0, ROWS_PT)])
        for i in range(K):
            for j in range(D // 16):
                rows0[i, pl.ds(j * 16, 16)] = jnp.ones((16,), jnp.float32)
        plsc.subcore_barrier()

        def deg_chunk(j, carry):
            pltpu.sync_copy(dst_hbm.at[pl.ds(ebase + j * K, K)], didx0)
            pltpu.sync_copy(rows0, agg_sh.at[didx0], add=True)
            return carry

        lax.fori_loop(0, CHUNKS, deg_chunk, 0)
        plsc.subcore_barrier()
        pltpu.sync_copy(agg_sh.at[pl.ds(r0, ROWS_PT)],
                        deg_out.at[cid, pl.ds(r0, ROWS_PT)])


def _make_sc_agg(with_deg):
    mesh = plsc.VectorSubcoreMesh(core_axis_name="c", subcore_axis_name="s")
    out_type = (jax.ShapeDtypeStruct((NC, N_PAD, D), jnp.float32),
                jax.ShapeDtypeStruct((NC, N_PAD, D), jnp.float32))
    scratch = [
        pltpu.VMEM_SHARED((N_PAD, D), jnp.float32),
        pltpu.VMEM((K,), jnp.int32),
        pltpu.VMEM((K,), jnp.int32),
        pltpu.VMEM((K, D), jnp.float32),
        pltpu.VMEM((K,), jnp.int32),
        pltpu.VMEM((K,), jnp.int32),
        pltpu.VMEM((K, D), jnp.float32),
        pltpu.SemaphoreType.DMA,
        pltpu.SemaphoreType.DMA,
    ]
    return pl.kernel(
        functools.partial(_sc_agg_body, with_deg),
        out_type=out_type,
        mesh=mesh,
        scratch_types=scratch,
        name="sc_edge_agg" + ("_deg" if with_deg else ""),
    )


_sc_agg_with_deg = _make_sc_agg(True)
_sc_agg_no_deg = _make_sc_agg(False)


def _tc_conv_body(h_ref, p_ref, degp_ref, ws_ref, wn_ref, b_ref, g_ref,
                  bb_ref, out_ref):
    deg = degp_ref[0, :N, 0:1] + degp_ref[1, :N, 0:1]
    inv = 1.0 / jnp.maximum(deg, 1.0)
    agg = (p_ref[0, :N, :] + p_ref[1, :N, :]) * inv
    y = (jnp.dot(h_ref[...], ws_ref[...], preferred_element_type=jnp.float32)
         + jnp.dot(agg, wn_ref[...], preferred_element_type=jnp.float32)
         + b_ref[...])
    m = jnp.mean(y, axis=0, keepdims=True)
    v = jnp.mean((y - m) ** 2, axis=0, keepdims=True)
    yn = (y - m) * lax.rsqrt(v + EPS) * g_ref[...] + bb_ref[...]
    out_ref[...] = jnp.maximum(yn, 0.0)


def _tc_conv(h, p, degp, ws, wn, b, g, bb):
    return pl.pallas_call(
        _tc_conv_body,
        out_shape=jax.ShapeDtypeStruct((N, D), jnp.float32),
    )(h, p, degp, ws, wn, b, g, bb)


def _tc_rest_body(h_ref, p_ref, degp_ref, ws_ref, wn_ref, b_ref, g_ref,
                  bb_ref, wl0_ref, bl0_ref, lg0_ref, lb0_ref, wl1_ref,
                  bl1_ref, lg1_ref, lb1_ref, wh_ref, bh_ref, mask_ref,
                  out_ref):
    deg = degp_ref[0, :N, 0:1] + degp_ref[1, :N, 0:1]
    inv = 1.0 / jnp.maximum(deg, 1.0)
    agg = (p_ref[0, :N, :] + p_ref[1, :N, :]) * inv
    y = (jnp.dot(h_ref[...], ws_ref[...], preferred_element_type=jnp.float32)
         + jnp.dot(agg, wn_ref[...], preferred_element_type=jnp.float32)
         + b_ref[...])
    m = jnp.mean(y, axis=0, keepdims=True)
    v = jnp.mean((y - m) ** 2, axis=0, keepdims=True)
    z = jnp.maximum((y - m) * lax.rsqrt(v + EPS) * g_ref[...] + bb_ref[...],
                    0.0)

    def mlp(t, wl, bl, lg, lb):
        u = jnp.dot(t, wl, preferred_element_type=jnp.float32) + bl
        mu = jnp.mean(u, axis=1, keepdims=True)
        var = jnp.mean((u - mu) ** 2, axis=1, keepdims=True)
        return jnp.maximum((u - mu) * lax.rsqrt(var + EPS) * lg + lb, 0.0)

    z = mlp(z, wl0_ref[...], bl0_ref[...], lg0_ref[...], lb0_ref[...])
    z = mlp(z, wl1_ref[...], bl1_ref[...], lg1_ref[...], lb1_ref[...])
    o = jnp.dot(z, wh_ref[...], preferred_element_type=jnp.float32) + bh_ref[...]
    out_ref[...] = o * mask_ref[...]


def _tc_rest(h, p, degp, ws, wn, b, g, bb, wl0, bl0, lg0, lb0, wl1, bl1,
             lg1, lb1, wh, bh, mask):
    return pl.pallas_call(
        _tc_rest_body,
        out_shape=jax.ShapeDtypeStruct((N, 1), jnp.float32),
    )(h, p, degp, ws, wn, b, g, bb, wl0, bl0, lg0, lb0, wl1, bl1, lg1, lb1,
      wh, bh, mask)


def kernel(x, edge_index, regression_mask,
           W_self0, W_neigh0, b_conv0, bn_g0, bn_b0,
           W_self1, W_neigh1, b_conv1, bn_g1, bn_b1,
           W_lin0, b_lin0, ln_g0, ln_b0,
           W_lin1, b_lin1, ln_g1, ln_b1,
           W_head, b_head):
    src = edge_index[0]
    dst = edge_index[1]
    z128 = jnp.zeros((N_PAD, D), dtype=jnp.float32)
    maskf = regression_mask.astype(jnp.float32).reshape(N, 1)

    p0, degp = _sc_agg_with_deg(x, src, dst, z128)
    h1 = _tc_conv(x, p0, degp,
                  W_self0, W_neigh0, b_conv0.reshape(1, D),
                  bn_g0.reshape(1, D), bn_b0.reshape(1, D))
    p1, _ = _sc_agg_no_deg(h1, src, dst, z128)
    out = _tc_rest(h1, p1, degp,
                   W_self1, W_neigh1, b_conv1.reshape(1, D),
                   bn_g1.reshape(1, D), bn_b1.reshape(1, D),
                   W_lin0, b_lin0.reshape(1, D), ln_g0.reshape(1, D),
                   ln_b0.reshape(1, D),
                   W_lin1, b_lin1.reshape(1, D), ln_g1.reshape(1, D),
                   ln_b1.reshape(1, D),
                   W_head, b_head.reshape(1, 1), maskf)
    return out[:, 0]
